# baseline (speedup 1.0000x reference)
import functools

import jax
import jax.numpy as jnp
from jax import lax
from jax.experimental import pallas as pl
from jax.experimental.pallas import tpu as pltpu
from jax.experimental.pallas import tpu_sc as plsc

N = 10000
E = 320000
D = 128
NCLS = 2

NC = 2
NS = 16
NW = NC * NS
L = 16

NPAD = 10240
EW = E // NW
SLC = NPAD // NS
CH = 125
NCH = EW // CH
SCH = 16
NSEG = NCH // SCH
STRIPE = NPAD // NS

@functools.cache
def _mesh():
    return plsc.VectorSubcoreMesh(core_axis_name="c", subcore_axis_name="s")


@functools.cache
def _deg_call():
    return pl.kernel(
        _deg,
        mesh=_mesh(),
        out_type=jax.ShapeDtypeStruct((NC, 2, NPAD), jnp.float32),
        scratch_types=[
            pltpu.VMEM((2, EW), jnp.int32),
            pltpu.VMEM((NPAD,), jnp.float32),
            pltpu.VMEM((NPAD,), jnp.float32),
            pltpu.VMEM((NS, 2, SLC), jnp.float32),
            pltpu.VMEM((2, SLC), jnp.float32),
            pltpu.VMEM_SHARED((NS, 2, NPAD), jnp.float32),
        ],
        compiler_params=pltpu.CompilerParams(needs_layout_passes=False),
    )


def _deg(e2_hbm, out_hbm, idx_v, hs, hd, red_v, acc_v, shared):
    c = lax.axis_index("c")
    s = lax.axis_index("s")
    w = s * NC + c
    pltpu.sync_copy(e2_hbm.at[0, w], idx_v.at[0])
    pltpu.sync_copy(e2_hbm.at[1, w], idx_v.at[1])

    zeros = jnp.zeros((L,), jnp.float32)

    @pl.loop(0, NPAD, step=L)
    def _(k):
        hs[pl.ds(k, L)] = zeros
        hd[pl.ds(k, L)] = zeros

    ones = jnp.ones((L,), jnp.float32)

    @pl.loop(0, EW, step=L)
    def _(j):
        plsc.addupdate_scatter(hs, [idx_v[0, pl.ds(j, L)]], ones)
        plsc.addupdate_scatter(hd, [idx_v[1, pl.ds(j, L)]], ones)

    pltpu.sync_copy(hs, shared.at[s, 0])
    pltpu.sync_copy(hd, shared.at[s, 1])
    plsc.subcore_barrier()

    base = s * SLC
    for p in range(NS):
        pltpu.sync_copy(shared.at[p, :, pl.ds(base, SLC)], red_v.at[p])

    @pl.loop(0, SLC, step=L)
    def _(k):
        for e in range(2):
            a = red_v[0, e, pl.ds(k, L)]
            for p in range(1, NS):
                a = a + red_v[p, e, pl.ds(k, L)]
            acc_v[e, pl.ds(k, L)] = a

    pltpu.sync_copy(acc_v.at[0], out_hbm.at[c, 0, pl.ds(base, SLC)])
    pltpu.sync_copy(acc_v.at[1], out_hbm.at[c, 1, pl.ds(base, SLC)])


def _scale_body(hist_ref, x_ref, h_ref, s_ref):
    deg_out = hist_ref[0, 0] + hist_ref[1, 0]
    deg_in = hist_ref[0, 1] + hist_ref[1, 1]
    so = lax.rsqrt(jnp.maximum(deg_out, 1.0))
    si = lax.rsqrt(jnp.maximum(deg_in, 1.0))
    h_ref[...] = x_ref[...] * so
    s_ref[...] = si


def _scale(hist_col, x_pad):
    gb = NPAD // 8
    return pl.pallas_call(
        _scale_body,
        grid=(8,),
        in_specs=[
            pl.BlockSpec((2, 2, gb, 1), lambda i: (0, 0, i, 0)),
            pl.BlockSpec((gb, D), lambda i: (i, 0)),
        ],
        out_specs=[
            pl.BlockSpec((gb, D), lambda i: (i, 0)),
            pl.BlockSpec((gb, 1), lambda i: (i, 0)),
        ],
        out_shape=[
            jax.ShapeDtypeStruct((NPAD, D), jnp.float32),
            jax.ShapeDtypeStruct((NPAD, 1), jnp.float32),
        ],
    )(hist_col, x_pad)


@functools.cache
def _agg_call():
    return pl.kernel(
        _agg,
        mesh=_mesh(),
        out_type=jax.ShapeDtypeStruct((NC, NPAD, D), jnp.float32),
        scratch_types=[
            pltpu.VMEM((2, SCH, CH), jnp.int32),
            pltpu.VMEM((CH, D), jnp.float32),
            pltpu.VMEM((CH, D), jnp.float32),
            pltpu.VMEM_SHARED((NPAD, D), jnp.float32),
            pltpu.SemaphoreType.DMA,
            pltpu.SemaphoreType.DMA,
            pltpu.SemaphoreType.DMA,
            pltpu.SemaphoreType.DMA,
        ],
        compiler_params=pltpu.CompilerParams(needs_layout_passes=False),
    )


def _agg(h_hbm, e3_hbm, out_hbm, idx_v, ra, rb, agg, sema, semb, semc, semd):
    c = lax.axis_index("c")
    s = lax.axis_index("s")
    w = s * NC + c

    zeros = jnp.zeros((L,), jnp.float32)

    @pl.loop(0, CH)
    def _(i):
        for k in range(D // L):
            ra[i, pl.ds(k * L, L)] = zeros

    base = s * STRIPE
    for m in range(STRIPE // CH):
        pltpu.sync_copy(ra, agg.at[pl.ds(base + m * CH, CH)])
    pltpu.sync_copy(
        ra.at[pl.ds(0, STRIPE % CH)],
        agg.at[pl.ds(base + (STRIPE // CH) * CH, STRIPE % CH)],
    )
    plsc.subcore_barrier()

    for seg in range(NSEG):
        pltpu.sync_copy(e3_hbm.at[0, w, seg], idx_v.at[0])
        pltpu.sync_copy(e3_hbm.at[1, w, seg], idx_v.at[1])
        pltpu.async_copy(h_hbm.at[idx_v.at[0, 0]], ra, sema)
        pltpu.async_copy(h_hbm.at[idx_v.at[0, 1]], rb, semb)

        @pl.loop(0, SCH, step=2)
        def _(j):
            pltpu.make_async_copy(h_hbm.at[idx_v.at[0, 0]], ra, sema).wait()
            pltpu.async_copy(ra, agg.at[idx_v.at[1, j]], semc, add=True)
            pltpu.make_async_copy(h_hbm.at[idx_v.at[0, 0]], rb, semb).wait()
            pltpu.async_copy(rb, agg.at[idx_v.at[1, j + 1]], semd, add=True)

            pltpu.make_async_copy(ra, agg.at[idx_v.at[1, j]], semc).wait()

            @pl.when(j + 2 < SCH)
            def _():
                pltpu.async_copy(h_hbm.at[idx_v.at[0, j + 2]], ra, sema)

            pltpu.make_async_copy(rb, agg.at[idx_v.at[1, j + 1]], semd).wait()

            @pl.when(j + 3 < SCH)
            def _():
                pltpu.async_copy(h_hbm.at[idx_v.at[0, j + 3]], rb, semb)

    plsc.subcore_barrier()
    pltpu.sync_copy(agg.at[pl.ds(base, STRIPE)], out_hbm.at[c, pl.ds(base, STRIPE)])


def _head_body(aggp_ref, s_ref, w1_ref, b1_ref, wc1_ref, bc1_ref, wc2_ref,
               bc2_ref, out_ref, hg_ref):
    i = pl.program_id(0)
    rb = aggp_ref.shape[1]
    a = (aggp_ref[0] + aggp_ref[1]) * s_ref[...]
    t = jnp.maximum(
        jnp.dot(a, w1_ref[...], preferred_element_type=jnp.float32) + b1_ref[...],
        0.0,
    )
    row = lax.broadcasted_iota(jnp.int32, (rb, 1), 0) + i * rb
    t = jnp.where(row < N, t, 0.0)
    part = jnp.sum(t, axis=0, keepdims=True)

    @pl.when(i == 0)
    def _():
        hg_ref[...] = part

    @pl.when(i > 0)
    def _():
        hg_ref[...] = hg_ref[...] + part

    hg = hg_ref[...]
    a2 = jnp.dot(hg, wc1_ref[...], preferred_element_type=jnp.float32) + bc1_ref[...]
    a3 = jnp.dot(a2, wc2_ref[...], preferred_element_type=jnp.float32) + bc2_ref[...]
    out_ref[...] = jax.nn.sigmoid(a3)


def _head(aggp, in_scale, w1, b1, wc1, bc1, wc2p, bc2p):
    rb = NPAD // 8
    return pl.pallas_call(
        _head_body,
        grid=(8,),
        in_specs=[
            pl.BlockSpec((2, rb, D), lambda i: (0, i, 0)),
            pl.BlockSpec((rb, 1), lambda i: (i, 0)),
            pl.BlockSpec((D, D), lambda i: (0, 0)),
            pl.BlockSpec((1, D), lambda i: (0, 0)),
            pl.BlockSpec((D, D), lambda i: (0, 0)),
            pl.BlockSpec((1, D), lambda i: (0, 0)),
            pl.BlockSpec((D, D), lambda i: (0, 0)),
            pl.BlockSpec((1, D), lambda i: (0, 0)),
        ],
        out_specs=[
            pl.BlockSpec((1, D), lambda i: (0, 0)),
            pl.BlockSpec((1, D), lambda i: (0, 0)),
        ],
        out_shape=[
            jax.ShapeDtypeStruct((1, D), jnp.float32),
            jax.ShapeDtypeStruct((1, D), jnp.float32),
        ],
    )(aggp, in_scale, w1, b1, wc1, bc1, wc2p, bc2p)


def kernel(x, edge_index, W1, b1, Wc1, bc1, Wc2, bc2):
    e2 = edge_index.reshape(2, NW, EW)
    e3 = edge_index.reshape(2, NW, NSEG, SCH, CH)
    x_pad = jnp.pad(x, ((0, NPAD - N), (0, 0)))

    hist = _deg_call()(e2)
    hist_col = hist.reshape(NC, 2, NPAD, 1)
    h, in_scale = _scale(hist_col, x_pad)
    aggp = _agg_call()(h, e3)

    wc2p = jnp.pad(Wc2, ((0, 0), (0, D - NCLS)))
    bc2p = jnp.pad(bc2, (0, D - NCLS)).reshape(1, D)
    out_full, hg = _head(
        aggp, in_scale, W1, b1.reshape(1, D), Wc1, bc1.reshape(1, D), wc2p, bc2p
    )
    out = out_full[:, :NCLS]
    return (out, hg, hg)


# speedup vs baseline: 1.1860x; 1.1860x over previous
import functools

import jax
import jax.numpy as jnp
from jax import lax
from jax.experimental import pallas as pl
from jax.experimental.pallas import tpu as pltpu
from jax.experimental.pallas import tpu_sc as plsc

N = 10000
E = 320000
D = 128
NCLS = 2

NC = 2
NS = 16
NW = NC * NS
L = 16

NPAD = 10240
EW = E // NW
SLC = NPAD // NS
CH = 125
NCH = EW // CH
SCH = 16
NSEG = NCH // SCH
STRIPE = NPAD // NS

@functools.cache
def _mesh():
    return plsc.VectorSubcoreMesh(core_axis_name="c", subcore_axis_name="s")


@functools.cache
def _deg_call():
    return pl.kernel(
        _deg,
        mesh=_mesh(),
        out_type=jax.ShapeDtypeStruct((NC, 2, NPAD), jnp.float32),
        scratch_types=[
            pltpu.VMEM((2, EW), jnp.int32),
            pltpu.VMEM((NPAD,), jnp.float32),
            pltpu.VMEM((NPAD,), jnp.float32),
            pltpu.VMEM((NS, 2, SLC), jnp.float32),
            pltpu.VMEM((2, SLC), jnp.float32),
            pltpu.VMEM_SHARED((NS, 2, NPAD), jnp.float32),
        ],
        compiler_params=pltpu.CompilerParams(needs_layout_passes=False),
    )


def _deg(e2_hbm, out_hbm, idx_v, hs, hd, red_v, acc_v, shared):
    c = lax.axis_index("c")
    s = lax.axis_index("s")
    w = s * NC + c
    pltpu.sync_copy(e2_hbm.at[0, w], idx_v.at[0])
    pltpu.sync_copy(e2_hbm.at[1, w], idx_v.at[1])

    zeros = jnp.zeros((L,), jnp.float32)

    @pl.loop(0, NPAD, step=L, unroll=8)
    def _(k):
        hs[pl.ds(k, L)] = zeros
        hd[pl.ds(k, L)] = zeros

    ones = jnp.ones((L,), jnp.float32)

    @pl.loop(0, EW, step=L, unroll=5)
    def _(j):
        plsc.addupdate_scatter(hs, [idx_v[0, pl.ds(j, L)]], ones)
        plsc.addupdate_scatter(hd, [idx_v[1, pl.ds(j, L)]], ones)

    pltpu.sync_copy(hs, shared.at[s, 0])
    pltpu.sync_copy(hd, shared.at[s, 1])
    plsc.subcore_barrier()

    base = s * SLC
    for p in range(NS):
        pltpu.sync_copy(shared.at[p, :, pl.ds(base, SLC)], red_v.at[p])

    @pl.loop(0, SLC, step=L)
    def _(k):
        for e in range(2):
            a = red_v[0, e, pl.ds(k, L)]
            for p in range(1, NS):
                a = a + red_v[p, e, pl.ds(k, L)]
            acc_v[e, pl.ds(k, L)] = a

    pltpu.sync_copy(acc_v.at[0], out_hbm.at[c, 0, pl.ds(base, SLC)])
    pltpu.sync_copy(acc_v.at[1], out_hbm.at[c, 1, pl.ds(base, SLC)])


def _scale_body(hist_ref, x_ref, h_ref, s_ref):
    deg_out = hist_ref[0, 0] + hist_ref[1, 0]
    deg_in = hist_ref[0, 1] + hist_ref[1, 1]
    so = lax.rsqrt(jnp.maximum(deg_out, 1.0))
    si = lax.rsqrt(jnp.maximum(deg_in, 1.0))
    h_ref[...] = x_ref[...] * so
    s_ref[...] = si


def _scale(hist_col, x):
    gb = N // 10
    return pl.pallas_call(
        _scale_body,
        grid=(10,),
        in_specs=[
            pl.BlockSpec((2, 2, gb, 1), lambda i: (0, 0, i, 0)),
            pl.BlockSpec((gb, D), lambda i: (i, 0)),
        ],
        out_specs=[
            pl.BlockSpec((gb, D), lambda i: (i, 0)),
            pl.BlockSpec((gb, 1), lambda i: (i, 0)),
        ],
        out_shape=[
            jax.ShapeDtypeStruct((N, D), jnp.float32),
            jax.ShapeDtypeStruct((N, 1), jnp.float32),
        ],
    )(hist_col, x)


@functools.cache
def _agg_call():
    return pl.kernel(
        _agg,
        mesh=_mesh(),
        out_type=jax.ShapeDtypeStruct((NC, NPAD, D), jnp.float32),
        scratch_types=[
            pltpu.VMEM((2, SCH, CH), jnp.int32),
            pltpu.VMEM((CH, D), jnp.float32),
            pltpu.VMEM((CH, D), jnp.float32),
            pltpu.VMEM_SHARED((NPAD, D), jnp.float32),
            pltpu.SemaphoreType.DMA,
            pltpu.SemaphoreType.DMA,
        ],
        compiler_params=pltpu.CompilerParams(needs_layout_passes=False),
    )


def _agg(h_hbm, e3_hbm, out_hbm, idx_v, ra, rb, agg, sema, semb):
    c = lax.axis_index("c")
    s = lax.axis_index("s")
    w = s * NC + c

    zeros = jnp.zeros((L,), jnp.float32)

    @pl.loop(0, CH)
    def _(i):
        for k in range(D // L):
            ra[i, pl.ds(k * L, L)] = zeros

    base = s * STRIPE
    for m in range(STRIPE // CH):
        pltpu.sync_copy(ra, agg.at[pl.ds(base + m * CH, CH)])
    pltpu.sync_copy(
        ra.at[pl.ds(0, STRIPE % CH)],
        agg.at[pl.ds(base + (STRIPE // CH) * CH, STRIPE % CH)],
    )
    plsc.subcore_barrier()

    for seg in range(NSEG):
        pltpu.sync_copy(e3_hbm.at[0, w, seg], idx_v.at[0])
        pltpu.sync_copy(e3_hbm.at[1, w, seg], idx_v.at[1])
        pltpu.async_copy(h_hbm.at[idx_v.at[0, 0]], ra, sema)

        @pl.loop(0, SCH, step=2)
        def _(j):
            pltpu.async_copy(h_hbm.at[idx_v.at[0, j + 1]], rb, semb)
            pltpu.make_async_copy(h_hbm.at[idx_v.at[0, 0]], ra, sema).wait()
            pltpu.sync_copy(ra, agg.at[idx_v.at[1, j]], add=True)

            @pl.when(j + 2 < SCH)
            def _():
                pltpu.async_copy(h_hbm.at[idx_v.at[0, j + 2]], ra, sema)

            pltpu.make_async_copy(h_hbm.at[idx_v.at[0, 0]], rb, semb).wait()
            pltpu.sync_copy(rb, agg.at[idx_v.at[1, j + 1]], add=True)

    plsc.subcore_barrier()
    pltpu.sync_copy(agg.at[pl.ds(base, STRIPE)], out_hbm.at[c, pl.ds(base, STRIPE)])


def _head_body(aggp_ref, s_ref, w1_ref, b1_ref, wc1_ref, bc1_ref, wc2_ref,
               bc2_ref, out_ref, hg_ref):
    i = pl.program_id(0)
    a = (aggp_ref[0] + aggp_ref[1]) * s_ref[...]
    t = jnp.maximum(
        jnp.dot(a, w1_ref[...], preferred_element_type=jnp.float32) + b1_ref[...],
        0.0,
    )
    part = jnp.sum(t, axis=0, keepdims=True)

    @pl.when(i == 0)
    def _():
        hg_ref[...] = part

    @pl.when(i > 0)
    def _():
        hg_ref[...] = hg_ref[...] + part

    hg = hg_ref[...]
    a2 = jnp.dot(hg, wc1_ref[...], preferred_element_type=jnp.float32) + bc1_ref[...]
    a3 = jnp.dot(a2, wc2_ref[...], preferred_element_type=jnp.float32) + bc2_ref[...]
    out_ref[...] = jax.nn.sigmoid(a3)


def _head(aggp, in_scale, w1, b1, wc1, bc1, wc2p, bc2p):
    rb = N // 5
    return pl.pallas_call(
        _head_body,
        grid=(5,),
        in_specs=[
            pl.BlockSpec((2, rb, D), lambda i: (0, i, 0)),
            pl.BlockSpec((rb, 1), lambda i: (i, 0)),
            pl.BlockSpec((D, D), lambda i: (0, 0)),
            pl.BlockSpec((1, D), lambda i: (0, 0)),
            pl.BlockSpec((D, D), lambda i: (0, 0)),
            pl.BlockSpec((1, D), lambda i: (0, 0)),
            pl.BlockSpec((D, D), lambda i: (0, 0)),
            pl.BlockSpec((1, D), lambda i: (0, 0)),
        ],
        out_specs=[
            pl.BlockSpec((1, D), lambda i: (0, 0)),
            pl.BlockSpec((1, D), lambda i: (0, 0)),
        ],
        out_shape=[
            jax.ShapeDtypeStruct((1, D), jnp.float32),
            jax.ShapeDtypeStruct((1, D), jnp.float32),
        ],
    )(aggp, in_scale, w1, b1, wc1, bc1, wc2p, bc2p)


def kernel(x, edge_index, W1, b1, Wc1, bc1, Wc2, bc2):
    e2 = edge_index.reshape(2, NW, EW)
    e3 = edge_index.reshape(2, NW, NSEG, SCH, CH)

    hist = _deg_call()(e2)
    hist_col = hist.reshape(NC, 2, NPAD, 1)
    h, in_scale = _scale(hist_col, x)
    aggp = _agg_call()(h, e3)

    wc2p = jnp.pad(Wc2, ((0, 0), (0, D - NCLS)))
    bc2p = jnp.pad(bc2, (0, D - NCLS)).reshape(1, D)
    out_full, hg = _head(
        aggp, in_scale, W1, b1.reshape(1, D), Wc1, bc1.reshape(1, D), wc2p, bc2p
    )
    out = out_full[:, :NCLS]
    return (out, hg, hg)
